# baseline (speedup 1.0000x reference)
import functools

import jax
import jax.numpy as jnp
from jax import lax
from jax.experimental import pallas as pl
from jax.experimental.pallas import tpu as pltpu
from jax.experimental.pallas import tpu_sc as plsc

N = 10000
E = 160000
SND = 53
TQ = 20
HID = 64
CH = 256
HEADS = 8
HC = 32

NSUB = 16
NCORE = 2
CHUNK = 125
SC_CHUNKS = E // NSUB // CHUNK
G_CHUNKS = E // (NSUB * NCORE) // CHUNK
ROWS_SUB = N // NSUB

BLK_E = 2000
BLK_L = 800

GM_CFG = ((0, 0, 0), (1, 0, 0), (2, 1, 1), (1, 1, 1))
CV_CFG = ((0, 0, 0), (2, 0, 0), (1, 0, 1))

_F32 = jnp.float32
_BF16 = jnp.bfloat16


def _hl(x):
    hi = x.astype(_BF16)
    lo = (x - hi.astype(_F32)).astype(_BF16)
    return hi, lo


def _dotp(a, b):
    return jnp.dot(a, b, preferred_element_type=_F32)


def _dot3(a, b):
    ah, al = _hl(a)
    bh, bl = _hl(b)
    return _dotp(ah, bh) + (_dotp(ah, bl) + _dotp(al, bh))


def _dot2r(a, b):
    ah = a.astype(_BF16)
    bh, bl = _hl(b)
    return _dotp(ah, bh) + _dotp(ah, bl)


def _dot2l(a, b):
    ah, al = _hl(a)
    bh = b.astype(_BF16)
    return _dotp(ah, bh) + _dotp(al, bh)


def _lstm_step(oh, h, c, ptab, whh):
    g = _dot2r(oh, ptab) + _dot3(h, whh)
    gi, gf, gg, go = jnp.split(g, 4, axis=1)
    c = jax.nn.sigmoid(gf) * c + jax.nn.sigmoid(gi) * jnp.tanh(gg)
    h = jax.nn.sigmoid(go) * jnp.tanh(c)
    return h, c


def _lstm_body(tok_ref, er_ref, emb_ref, pwf_ref, whh_ref, bf_ref, pwb_ref,
               whb_ref, bb_ref, regex_ref, gmsg_ref):
    tok = tok_ref[...]
    emb = emb_ref[...]
    pf = jnp.dot(emb, pwf_ref[...], preferred_element_type=_F32, precision=jax.lax.Precision.HIGHEST) + bf_ref[...]
    pb = jnp.dot(emb, pwb_ref[...], preferred_element_type=_F32, precision=jax.lax.Precision.HIGHEST) + bb_ref[...]
    whh = whh_ref[...]
    whb = whb_ref[...]
    ohs = [(tok[:, t:t + 1] ==
            lax.broadcasted_iota(jnp.int32, (BLK_L, 32), 1)).astype(_F32)
           for t in range(TQ)]
    h = jnp.zeros((BLK_L, HID), _F32)
    c = jnp.zeros((BLK_L, HID), _F32)
    hb = jnp.zeros((BLK_L, HID), _F32)
    cb = jnp.zeros((BLK_L, HID), _F32)
    for t in range(TQ):
        h, c = _lstm_step(ohs[t], h, c, pf, whh)
        hb, cb = _lstm_step(ohs[TQ - 1 - t], hb, cb, pb, whb)
    regex = jnp.concatenate([h, hb], axis=1)
    regex_ref[...] = regex
    er = er_ref[...]
    one = jnp.ones((BLK_L, 1), _F32)
    z74 = jnp.zeros((BLK_L, 74), _F32)
    gmsg_ref[0] = jnp.concatenate([er[:, SND:2 * SND], one, z74], axis=1)
    gmsg_ref[1] = regex
    gmsg_ref[2] = jnp.concatenate([er[:, 0:SND], one, z74], axis=1)


def _lstm_call(tokens, edge_rest, emb_pad, pwf, whh, bf, pwb, whb, bb):
    full = lambda r, c: pl.BlockSpec((r, c), lambda i: (0, 0))
    return pl.pallas_call(
        _lstm_body,
        grid=(E // BLK_L,),
        in_specs=[
            pl.BlockSpec((BLK_L, TQ), lambda i: (i, 0)),
            pl.BlockSpec((BLK_L, 2 * SND), lambda i: (i, 0)),
            full(32, 32), full(32, 256), full(64, 256), full(1, 256),
            full(32, 256), full(64, 256), full(1, 256),
        ],
        out_specs=[pl.BlockSpec((BLK_L, 128), lambda i: (i, 0)),
                   pl.BlockSpec((3, BLK_L, 128), lambda i: (0, i, 0))],
        out_shape=[jax.ShapeDtypeStruct((E, 128), _F32),
                   jax.ShapeDtypeStruct((3, E, 128), _F32)],
    )(tokens, edge_rest, emb_pad, pwf, whh, bf, pwb, whb, bb)


def _edge_body(xls_ref, xrd_ref, rgx_ref, we_ref, att_ref, out_ref, ex_ref):
    s1 = ((lax.broadcasted_iota(jnp.int32, (CH, HEADS), 0) // HC) ==
          lax.broadcasted_iota(jnp.int32, (CH, HEADS), 1)).astype(_F32)
    s2 = ((lax.broadcasted_iota(jnp.int32, (HEADS, CH), 1) // HC) ==
          lax.broadcasted_iota(jnp.int32, (HEADS, CH), 0)).astype(_F32)
    xls = xls_ref[...]
    ef = _dot3(rgx_ref[...], we_ref[...])
    z = xls + xrd_ref[...] + ef
    w = jnp.where(z > 0, z, 0.2 * z) * att_ref[...]
    a8 = _dot2l(w, s1)
    ex = jnp.exp(a8)
    exf = _dot2l(ex, s2)
    msg = xls * exf
    out_ref[0] = msg[:, :128]
    out_ref[1] = msg[:, 128:]
    ex_ref[...] = ex


def _edge_call(xls, xrd, regex, we, attf):
    full = lambda r, c: pl.BlockSpec((r, c), lambda i: (0, 0))
    return pl.pallas_call(
        _edge_body,
        grid=(E // BLK_E,),
        in_specs=[
            pl.BlockSpec((BLK_E, CH), lambda i: (i, 0)),
            pl.BlockSpec((BLK_E, CH), lambda i: (i, 0)),
            pl.BlockSpec((BLK_E, 128), lambda i: (i, 0)),
            full(128, CH), full(1, CH),
        ],
        out_specs=[pl.BlockSpec((2, BLK_E, 128), lambda i: (0, i, 0)),
                   pl.BlockSpec((BLK_E, 8), lambda i: (i, 0))],
        out_shape=[jax.ShapeDtypeStruct((2, E, 128), _F32),
                   jax.ShapeDtypeStruct((E, 8), _F32)],
    )(xls, xrd, regex, we, attf)


def _nd_to_x(nd, den, b):
    num = jnp.concatenate([nd[0], nd[1]], axis=1)
    s2 = ((lax.broadcasted_iota(jnp.int32, (HEADS, CH), 1) // HC) ==
          lax.broadcasted_iota(jnp.int32, (HEADS, CH), 0)).astype(_F32)
    denf = _dot2l(den, s2)
    o = num / (denf + 1e-16) + b
    return jnp.where(o > 0, o, jnp.exp(o) - 1.0)


def _finish_mm_body(nd_ref, den_ref, b_ref, wl_ref, wr_ref, xl_ref, xr_ref):
    x = _nd_to_x(nd_ref[...], den_ref[...], b_ref[...])
    xl_ref[...] = _dot3(x, wl_ref[...])
    xr_ref[...] = _dot3(x, wr_ref[...])


def _finish_last_body(nd_ref, den_ref, b_ref, x_ref):
    x_ref[...] = _nd_to_x(nd_ref[...], den_ref[...], b_ref[...])


BLK_N = 2000


def _finish_call(nd, nd8, b, wl=None, wr=None):
    full = lambda r, c: pl.BlockSpec((r, c), lambda i: (0, 0))
    nd_spec = pl.BlockSpec((2, BLK_N, 128), lambda i: (0, i, 0))
    den_spec = pl.BlockSpec((BLK_N, 8), lambda i: (i, 0))
    o_spec = pl.BlockSpec((BLK_N, CH), lambda i: (i, 0))
    o_shape = jax.ShapeDtypeStruct((N, CH), _F32)
    if wl is None:
        return pl.pallas_call(
            _finish_last_body, grid=(N // BLK_N,),
            in_specs=[nd_spec, den_spec, full(1, CH)],
            out_specs=o_spec, out_shape=o_shape,
        )(nd, nd8, b)
    return pl.pallas_call(
        _finish_mm_body, grid=(N // BLK_N,),
        in_specs=[nd_spec, den_spec, full(1, CH), full(CH, CH),
                  full(CH, CH)],
        out_specs=[o_spec, o_spec], out_shape=[o_shape, o_shape],
    )(nd, nd8, b, wl, wr)


def _prep0_body(gm_ref, lx_ref, rx_ref, wl_ref, wr_ref,
                oll_ref, olr_ref, orl_ref, orr_ref):
    gm = gm_ref[...]
    cs = jnp.maximum(gm[0][:, SND:SND + 1], 1.0)
    ct = jnp.maximum(gm[2][:, SND:SND + 1], 1.0)
    out_tr = jnp.concatenate([gm[0][:, 0:SND], gm[1]], axis=1) / cs
    in_tr = jnp.concatenate([gm[2][:, 0:SND], gm[3]], axis=1) / ct
    lxf = jnp.concatenate([lx_ref[...], in_tr, out_tr], axis=1)
    rxf = jnp.concatenate([rx_ref[...], out_tr, in_tr], axis=1)
    wl = wl_ref[...]
    wr = wr_ref[...]
    oll_ref[...] = _dot3(lxf, wl)
    olr_ref[...] = _dot3(lxf, wr)
    orl_ref[...] = _dot3(rxf, wl)
    orr_ref[...] = _dot3(rxf, wr)


def _prep0_call(gm, left_x, right_x, wl0, wr0):
    full = lambda r, c: pl.BlockSpec((r, c), lambda i: (0, 0))
    in_dim = wl0.shape[0]
    o_spec = pl.BlockSpec((BLK_N, CH), lambda i: (i, 0))
    o_shape = jax.ShapeDtypeStruct((N, CH), _F32)
    return pl.pallas_call(
        _prep0_body, grid=(N // BLK_N,),
        in_specs=[
            pl.BlockSpec((4, BLK_N, 128), lambda i: (0, i, 0)),
            pl.BlockSpec((BLK_N, SND + 2), lambda i: (i, 0)),
            pl.BlockSpec((BLK_N, SND + 2), lambda i: (i, 0)),
            full(in_dim, CH), full(in_dim, CH),
        ],
        out_specs=[o_spec] * 4, out_shape=[o_shape] * 4,
    )(gm, left_x, right_x, wl0, wr0)


def _make_scatter(cfg, nmsg, nkey):
    mesh = plsc.VectorSubcoreMesh(core_axis_name="c", subcore_axis_name="s")
    npass = len(cfg)

    @functools.partial(
        pl.kernel,
        out_type=jax.ShapeDtypeStruct((npass, N, 128), _F32),
        mesh=mesh,
        scratch_types=[
            pltpu.VMEM((SC_CHUNKS, CHUNK), jnp.int32),
            pltpu.VMEM((CHUNK, 128), _F32),
            pltpu.VMEM((CHUNK, 128), _F32),
            pltpu.VMEM_SHARED((N, 128), _F32),
            pltpu.SemaphoreType.DMA,
            pltpu.SemaphoreType.DMA,
        ],
        compiler_params=pltpu.CompilerParams(use_tc_tiling_on_sc=False),
    )
    def _scatter(msg_hbm, key_hbm, zer_hbm, out_hbm, idx_v, msg_a, msg_b,
                 acc, sem_a, sem_b):
        cid = lax.axis_index("c")
        sid = lax.axis_index("s")
        r0 = sid * ROWS_SUB
        e0 = sid * (E // NSUB)

        def _one_pass(mi, ki, p):
            pltpu.sync_copy(zer_hbm.at[pl.ds(r0, ROWS_SUB)],
                            acc.at[pl.ds(r0, ROWS_SUB)])
            pltpu.sync_copy(key_hbm.at[ki].at[sid], idx_v)
            plsc.subcore_barrier()

            mh = msg_hbm.at[mi]
            pltpu.async_copy(mh.at[pl.ds(e0, CHUNK)], msg_a, sem_a)

            def body(j2, carry):
                j = 2 * j2
                d1 = pltpu.async_copy(
                    mh.at[pl.ds(e0 + (j + 1) * CHUNK, CHUNK)], msg_b, sem_b)
                pltpu.make_async_copy(mh.at[pl.ds(0, CHUNK)], msg_a,
                                      sem_a).wait()
                pltpu.sync_copy(msg_a, acc.at[idx_v.at[j]], add=True)

                @pl.when(j2 + 1 < SC_CHUNKS // 2)
                def _():
                    pltpu.async_copy(
                        mh.at[pl.ds(e0 + (j + 2) * CHUNK, CHUNK)], msg_a,
                        sem_a)

                d1.wait()
                pltpu.sync_copy(msg_b, acc.at[idx_v.at[j + 1]], add=True)
                return carry

            lax.fori_loop(0, SC_CHUNKS // 2, body, 0)
            plsc.subcore_barrier()
            pltpu.sync_copy(acc.at[pl.ds(r0, ROWS_SUB)],
                            out_hbm.at[p].at[pl.ds(r0, ROWS_SUB)])
            plsc.subcore_barrier()

        for p, (mi, ki, co) in enumerate(cfg):
            pl.when(cid == co)(
                lambda mi=mi, ki=ki, p=p: _one_pass(mi, ki, p))

    return _scatter


@functools.lru_cache(maxsize=None)
def _get_scatter(kind):
    if kind == "gmp":
        return _make_scatter(GM_CFG, 3, 2)
    return _make_scatter(CV_CFG, 3, 1)


def _make_scatter_cv():
    mesh = plsc.VectorSubcoreMesh(core_axis_name="c", subcore_axis_name="s")

    @functools.partial(
        pl.kernel,
        out_type=(jax.ShapeDtypeStruct((2, N, 128), _F32),
                  jax.ShapeDtypeStruct((N, 8), _F32)),
        mesh=mesh,
        scratch_types=[
            pltpu.VMEM((SC_CHUNKS, CHUNK), jnp.int32),
            pltpu.VMEM((CHUNK, 128), _F32),
            pltpu.VMEM((CHUNK, 128), _F32),
            pltpu.VMEM((CHUNK, 8), _F32),
            pltpu.VMEM((CHUNK, 8), _F32),
            pltpu.VMEM_SHARED((N, 128), _F32),
            pltpu.VMEM_SHARED((N, 8), _F32),
            pltpu.SemaphoreType.DMA,
            pltpu.SemaphoreType.DMA,
        ],
        compiler_params=pltpu.CompilerParams(use_tc_tiling_on_sc=False),
    )
    def _scatter_cv(msg_hbm, ex_hbm, key_hbm, zer_hbm, out_hbm, out8_hbm,
                    idx_v, msg_a, msg_b, ex_a, ex_b, acc, acc8, sem_a, sem_b):
        cid = lax.axis_index("c")
        sid = lax.axis_index("s")
        r0 = sid * ROWS_SUB
        e0 = sid * (E // NSUB)

        def _pass(mh, buf_a, buf_b, my_acc, width, out_slice):
            pltpu.sync_copy(
                zer_hbm.at[pl.ds(r0, ROWS_SUB), pl.ds(0, width)],
                my_acc.at[pl.ds(r0, ROWS_SUB)])
            plsc.subcore_barrier()
            pltpu.async_copy(mh.at[pl.ds(e0, CHUNK)], buf_a, sem_a)

            def body(j2, carry):
                j = 2 * j2
                d1 = pltpu.async_copy(
                    mh.at[pl.ds(e0 + (j + 1) * CHUNK, CHUNK)], buf_b, sem_b)
                pltpu.make_async_copy(mh.at[pl.ds(0, CHUNK)], buf_a,
                                      sem_a).wait()
                pltpu.sync_copy(buf_a, my_acc.at[idx_v.at[j]], add=True)

                @pl.when(j2 + 1 < SC_CHUNKS // 2)
                def _():
                    pltpu.async_copy(
                        mh.at[pl.ds(e0 + (j + 2) * CHUNK, CHUNK)], buf_a,
                        sem_a)

                d1.wait()
                pltpu.sync_copy(buf_b, my_acc.at[idx_v.at[j + 1]], add=True)
                return carry

            lax.fori_loop(0, SC_CHUNKS // 2, body, 0)
            plsc.subcore_barrier()
            pltpu.sync_copy(my_acc.at[pl.ds(r0, ROWS_SUB)], out_slice)
            plsc.subcore_barrier()

        pltpu.sync_copy(key_hbm.at[sid], idx_v)

        @pl.when(cid == 0)
        def _():
            _pass(msg_hbm.at[0], msg_a, msg_b, acc, 128,
                  out_hbm.at[0].at[pl.ds(r0, ROWS_SUB)])
            _pass(ex_hbm, ex_a, ex_b, acc8, 8,
                  out8_hbm.at[pl.ds(r0, ROWS_SUB)])

        @pl.when(cid == 1)
        def _():
            _pass(msg_hbm.at[1], msg_a, msg_b, acc, 128,
                  out_hbm.at[1].at[pl.ds(r0, ROWS_SUB)])

    return _scatter_cv


@functools.lru_cache(maxsize=None)
def _get_scatter_cv():
    return _make_scatter_cv()


def _make_gather2():
    mesh = plsc.VectorSubcoreMesh(core_axis_name="c", subcore_axis_name="s")

    @functools.partial(
        pl.kernel,
        out_type=(jax.ShapeDtypeStruct((E, CH), _F32),
                  jax.ShapeDtypeStruct((E, CH), _F32)),
        mesh=mesh,
        scratch_types=[
            pltpu.VMEM((SC_CHUNKS, CHUNK), jnp.int32),
            pltpu.VMEM((CHUNK, CH), _F32),
            pltpu.VMEM((CHUNK, CH), _F32),
            pltpu.SemaphoreType.DMA,
            pltpu.SemaphoreType.DMA,
            pltpu.SemaphoreType.DMA,
            pltpu.SemaphoreType.DMA,
        ],
        compiler_params=pltpu.CompilerParams(use_tc_tiling_on_sc=False),
    )
    def _gather2(xl_hbm, xr_hbm, sidx_hbm, didx_hbm, oxl_hbm, oxr_hbm,
                 idx_v, ra_v, rb_v, gsem_a, gsem_b, osem_a, osem_b):
        cid = lax.axis_index("c")
        sid = lax.axis_index("s")
        e0 = sid * (E // NSUB)

        def _chain(tab_hbm, keys_hbm, out_hbm):
            pltpu.sync_copy(keys_hbm.at[sid], idx_v)

            def body(j2, carry):
                j = 2 * j2

                @pl.when(j2 > 0)
                def _():
                    pltpu.make_async_copy(
                        ra_v, out_hbm.at[pl.ds(e0, CHUNK)], osem_a).wait()
                    pltpu.make_async_copy(
                        rb_v, out_hbm.at[pl.ds(e0, CHUNK)], osem_b).wait()

                ga = pltpu.async_copy(tab_hbm.at[idx_v.at[j]], ra_v, gsem_a)
                gb = pltpu.async_copy(tab_hbm.at[idx_v.at[j + 1]], rb_v,
                                      gsem_b)
                ga.wait()
                pltpu.async_copy(ra_v, out_hbm.at[pl.ds(e0 + j * CHUNK,
                                                        CHUNK)], osem_a)
                gb.wait()
                pltpu.async_copy(rb_v, out_hbm.at[pl.ds(e0 + (j + 1) * CHUNK,
                                                        CHUNK)], osem_b)
                return carry

            lax.fori_loop(0, SC_CHUNKS // 2, body, 0)
            pltpu.make_async_copy(ra_v, out_hbm.at[pl.ds(e0, CHUNK)],
                                  osem_a).wait()
            pltpu.make_async_copy(rb_v, out_hbm.at[pl.ds(e0, CHUNK)],
                                  osem_b).wait()

        pl.when(cid == 0)(lambda: _chain(xl_hbm, sidx_hbm, oxl_hbm))
        pl.when(cid == 1)(lambda: _chain(xr_hbm, didx_hbm, oxr_hbm))

    return _gather2


@functools.lru_cache(maxsize=None)
def _get_gather2():
    return _make_gather2()


def _sc_gather2(xl, xr, sidx, didx):
    return _get_gather2()(xl, xr, sidx, didx)


def _sc_scatter(msg, keys, kind):
    zer = jnp.zeros((N, 128), _F32)
    return _get_scatter(kind)(msg, keys, zer)


def kernel(left_x, right_x, left_edge_index, right_edge_index, edge_tokens,
           edge_rest, embed_table, lstm_params, conv_params):
    emb_pad = jnp.zeros((32, 32), _F32).at[:30].set(embed_table)
    pwf = lstm_params['W_ih_f'].T
    whh = lstm_params['W_hh_f'].T
    bf = (lstm_params['b_ih_f'] + lstm_params['b_hh_f']).reshape(1, 256)
    pwb = lstm_params['W_ih_b'].T
    whb = lstm_params['W_hh_b'].T
    bb = (lstm_params['b_ih_b'] + lstm_params['b_hh_b']).reshape(1, 256)
    regex, gmsg = _lstm_call(edge_tokens, edge_rest, emb_pad, pwf, whh, bf,
                             pwb, whb, bb)

    s = left_edge_index[0]
    t = left_edge_index[1]
    gm_keys = jnp.stack([s.reshape(NSUB, SC_CHUNKS, CHUNK),
                         t.reshape(NSUB, SC_CHUNKS, CHUNK)], axis=0)
    gm = _sc_scatter(gmsg, gm_keys, "gmp")

    p0 = conv_params[0]
    xl_l, xr_l, xl_r, xr_r = _prep0_call(gm, left_x, right_x, p0['Wl'],
                                         p0['Wr'])

    def chain(xl, xr, edge_index):
        src = edge_index[0]
        dst = edge_index[1]
        sidx = src.reshape(NSUB, SC_CHUNKS, CHUNK)
        didx = dst.reshape(NSUB, SC_CHUNKS, CHUNK)
        zer = jnp.zeros((N, 128), _F32)
        for li, p in enumerate(conv_params):
            xls, xrd = _sc_gather2(xl, xr, sidx, didx)
            msg, ex = _edge_call(xls, xrd, regex, p['We'],
                                 p['att'].reshape(1, CH))
            nd, nd8 = _get_scatter_cv()(msg, ex, didx, zer)
            b = p['b'].reshape(1, CH)
            if li + 1 < len(conv_params):
                pn = conv_params[li + 1]
                xl, xr = _finish_call(nd, nd8, b, pn['Wl'], pn['Wr'])
            else:
                return _finish_call(nd, nd8, b)

    out_l = chain(xl_l, xr_l, left_edge_index)
    out_r = chain(xl_r, xr_r, right_edge_index)
    return jnp.concatenate([out_l, out_r], axis=-1)


# speedup vs baseline: 1.0165x; 1.0165x over previous
import functools

import jax
import jax.numpy as jnp
from jax import lax
from jax.experimental import pallas as pl
from jax.experimental.pallas import tpu as pltpu
from jax.experimental.pallas import tpu_sc as plsc

N = 10000
E = 160000
SND = 53
TQ = 20
HID = 64
CH = 256
HEADS = 8
HC = 32

NSUB = 16
NCORE = 2
CHUNK = 125
SC_CHUNKS = E // NSUB // CHUNK
G_CHUNKS = E // (NSUB * NCORE) // CHUNK
ROWS_SUB = N // NSUB

BLK_E = 2000
BLK_L = 800

GM_CFG = ((0, 0, 0), (1, 0, 0), (2, 1, 1), (1, 1, 1))
CV_CFG = ((0, 0, 0), (2, 0, 0), (1, 0, 1))

_F32 = jnp.float32
_BF16 = jnp.bfloat16


def _hl(x):
    hi = x.astype(_BF16)
    lo = (x - hi.astype(_F32)).astype(_BF16)
    return hi, lo


def _dotp(a, b):
    return jnp.dot(a, b, preferred_element_type=_F32)


def _dot3(a, b):
    ah, al = _hl(a)
    bh, bl = _hl(b)
    return _dotp(ah, bh) + (_dotp(ah, bl) + _dotp(al, bh))


def _dot2r(a, b):
    ah = a.astype(_BF16)
    bh, bl = _hl(b)
    return _dotp(ah, bh) + _dotp(ah, bl)


def _dot2l(a, b):
    ah, al = _hl(a)
    bh = b.astype(_BF16)
    return _dotp(ah, bh) + _dotp(al, bh)


def _lstm_step(oh, h, c, ptab, whh):
    g = _dot2r(oh, ptab) + _dot3(h, whh)
    gi, gf, gg, go = jnp.split(g, 4, axis=1)
    c = jax.nn.sigmoid(gf) * c + jax.nn.sigmoid(gi) * jnp.tanh(gg)
    h = jax.nn.sigmoid(go) * jnp.tanh(c)
    return h, c


def _lstm_body(tok_ref, er_ref, emb_ref, pwf_ref, whh_ref, bf_ref, pwb_ref,
               whb_ref, bb_ref, regex_ref, gmsg_ref):
    tok = tok_ref[...]
    emb = emb_ref[...]
    pf = jnp.dot(emb, pwf_ref[...], preferred_element_type=_F32, precision=jax.lax.Precision.HIGHEST) + bf_ref[...]
    pb = jnp.dot(emb, pwb_ref[...], preferred_element_type=_F32, precision=jax.lax.Precision.HIGHEST) + bb_ref[...]
    whh = whh_ref[...]
    whb = whb_ref[...]
    ohs = [(tok[:, t:t + 1] ==
            lax.broadcasted_iota(jnp.int32, (BLK_L, 32), 1)).astype(_F32)
           for t in range(TQ)]
    h = jnp.zeros((BLK_L, HID), _F32)
    c = jnp.zeros((BLK_L, HID), _F32)
    hb = jnp.zeros((BLK_L, HID), _F32)
    cb = jnp.zeros((BLK_L, HID), _F32)
    for t in range(TQ):
        h, c = _lstm_step(ohs[t], h, c, pf, whh)
        hb, cb = _lstm_step(ohs[TQ - 1 - t], hb, cb, pb, whb)
    regex = jnp.concatenate([h, hb], axis=1)
    regex_ref[...] = regex
    er = er_ref[...]
    one = jnp.ones((BLK_L, 1), _F32)
    z74 = jnp.zeros((BLK_L, 74), _F32)
    gmsg_ref[0] = jnp.concatenate([er[:, SND:2 * SND], one, z74], axis=1)
    gmsg_ref[1] = regex
    gmsg_ref[2] = jnp.concatenate([er[:, 0:SND], one, z74], axis=1)


def _lstm_call(tokens, edge_rest, emb_pad, pwf, whh, bf, pwb, whb, bb):
    full = lambda r, c: pl.BlockSpec((r, c), lambda i: (0, 0))
    return pl.pallas_call(
        _lstm_body,
        grid=(E // BLK_L,),
        in_specs=[
            pl.BlockSpec((BLK_L, TQ), lambda i: (i, 0)),
            pl.BlockSpec((BLK_L, 2 * SND), lambda i: (i, 0)),
            full(32, 32), full(32, 256), full(64, 256), full(1, 256),
            full(32, 256), full(64, 256), full(1, 256),
        ],
        out_specs=[pl.BlockSpec((BLK_L, 128), lambda i: (i, 0)),
                   pl.BlockSpec((3, BLK_L, 128), lambda i: (0, i, 0))],
        out_shape=[jax.ShapeDtypeStruct((E, 128), _F32),
                   jax.ShapeDtypeStruct((3, E, 128), _F32)],
    )(tokens, edge_rest, emb_pad, pwf, whh, bf, pwb, whb, bb)


def _edge_body(xls_ref, xrd_ref, rgx_ref, we_ref, att_ref, out_ref):
    s1 = ((lax.broadcasted_iota(jnp.int32, (CH, HEADS), 0) // HC) ==
          lax.broadcasted_iota(jnp.int32, (CH, HEADS), 1)).astype(_F32)
    s2 = ((lax.broadcasted_iota(jnp.int32, (HEADS, CH), 1) // HC) ==
          lax.broadcasted_iota(jnp.int32, (HEADS, CH), 0)).astype(_F32)
    xls = xls_ref[...]
    ef = _dot3(rgx_ref[...], we_ref[...])
    z = xls + xrd_ref[...] + ef
    w = jnp.where(z > 0, z, 0.2 * z) * att_ref[...]
    a8 = _dot2l(w, s1)
    ex = jnp.exp(a8)
    exf = _dot2l(ex, s2)
    msg = xls * exf
    z120 = jnp.zeros((BLK_E, 120), _F32)
    out_ref[0] = msg[:, :128]
    out_ref[1] = msg[:, 128:]
    out_ref[2] = jnp.concatenate([ex, z120], axis=1)


def _edge_call(xls, xrd, regex, we, attf):
    full = lambda r, c: pl.BlockSpec((r, c), lambda i: (0, 0))
    return pl.pallas_call(
        _edge_body,
        grid=(E // BLK_E,),
        in_specs=[
            pl.BlockSpec((BLK_E, CH), lambda i: (i, 0)),
            pl.BlockSpec((BLK_E, CH), lambda i: (i, 0)),
            pl.BlockSpec((BLK_E, 128), lambda i: (i, 0)),
            full(128, CH), full(1, CH),
        ],
        out_specs=pl.BlockSpec((3, BLK_E, 128), lambda i: (0, i, 0)),
        out_shape=jax.ShapeDtypeStruct((3, E, 128), _F32),
    )(xls, xrd, regex, we, attf)


def _nd_to_x(nd, b):
    num = jnp.concatenate([nd[0], nd[2]], axis=1)
    den = nd[1][:, 0:HEADS]
    s2 = ((lax.broadcasted_iota(jnp.int32, (HEADS, CH), 1) // HC) ==
          lax.broadcasted_iota(jnp.int32, (HEADS, CH), 0)).astype(_F32)
    denf = _dot2l(den, s2)
    o = num / (denf + 1e-16) + b
    return jnp.where(o > 0, o, jnp.exp(o) - 1.0)


def _finish_mm_body(nd_ref, b_ref, wl_ref, wr_ref, xl_ref, xr_ref):
    x = _nd_to_x(nd_ref[...], b_ref[...])
    xl_ref[...] = _dot3(x, wl_ref[...])
    xr_ref[...] = _dot3(x, wr_ref[...])


def _finish_last_body(nd_ref, b_ref, x_ref):
    x_ref[...] = _nd_to_x(nd_ref[...], b_ref[...])


BLK_N = 2000


def _finish_call(nd, b, wl=None, wr=None):
    full = lambda r, c: pl.BlockSpec((r, c), lambda i: (0, 0))
    nd_spec = pl.BlockSpec((3, BLK_N, 128), lambda i: (0, i, 0))
    o_spec = pl.BlockSpec((BLK_N, CH), lambda i: (i, 0))
    o_shape = jax.ShapeDtypeStruct((N, CH), _F32)
    if wl is None:
        return pl.pallas_call(
            _finish_last_body, grid=(N // BLK_N,),
            in_specs=[nd_spec, full(1, CH)],
            out_specs=o_spec, out_shape=o_shape,
        )(nd, b)
    return pl.pallas_call(
        _finish_mm_body, grid=(N // BLK_N,),
        in_specs=[nd_spec, full(1, CH), full(CH, CH), full(CH, CH)],
        out_specs=[o_spec, o_spec], out_shape=[o_shape, o_shape],
    )(nd, b, wl, wr)


def _prep0_body(gm_ref, lx_ref, rx_ref, wl_ref, wr_ref,
                oll_ref, olr_ref, orl_ref, orr_ref):
    gm = gm_ref[...]
    cs = jnp.maximum(gm[0][:, SND:SND + 1], 1.0)
    ct = jnp.maximum(gm[2][:, SND:SND + 1], 1.0)
    out_tr = jnp.concatenate([gm[0][:, 0:SND], gm[1]], axis=1) / cs
    in_tr = jnp.concatenate([gm[2][:, 0:SND], gm[3]], axis=1) / ct
    lxf = jnp.concatenate([lx_ref[...], in_tr, out_tr], axis=1)
    rxf = jnp.concatenate([rx_ref[...], out_tr, in_tr], axis=1)
    wl = wl_ref[...]
    wr = wr_ref[...]
    oll_ref[...] = _dot3(lxf, wl)
    olr_ref[...] = _dot3(lxf, wr)
    orl_ref[...] = _dot3(rxf, wl)
    orr_ref[...] = _dot3(rxf, wr)


def _prep0_call(gm, left_x, right_x, wl0, wr0):
    full = lambda r, c: pl.BlockSpec((r, c), lambda i: (0, 0))
    in_dim = wl0.shape[0]
    o_spec = pl.BlockSpec((BLK_N, CH), lambda i: (i, 0))
    o_shape = jax.ShapeDtypeStruct((N, CH), _F32)
    return pl.pallas_call(
        _prep0_body, grid=(N // BLK_N,),
        in_specs=[
            pl.BlockSpec((4, BLK_N, 128), lambda i: (0, i, 0)),
            pl.BlockSpec((BLK_N, SND + 2), lambda i: (i, 0)),
            pl.BlockSpec((BLK_N, SND + 2), lambda i: (i, 0)),
            full(in_dim, CH), full(in_dim, CH),
        ],
        out_specs=[o_spec] * 4, out_shape=[o_shape] * 4,
    )(gm, left_x, right_x, wl0, wr0)


def _make_scatter(cfg, nmsg, nkey):
    mesh = plsc.VectorSubcoreMesh(core_axis_name="c", subcore_axis_name="s")
    npass = len(cfg)

    @functools.partial(
        pl.kernel,
        out_type=jax.ShapeDtypeStruct((npass, N, 128), _F32),
        mesh=mesh,
        scratch_types=[
            pltpu.VMEM((SC_CHUNKS, CHUNK), jnp.int32),
            pltpu.VMEM((CHUNK, 128), _F32),
            pltpu.VMEM((CHUNK, 128), _F32),
            pltpu.VMEM_SHARED((N, 128), _F32),
            pltpu.SemaphoreType.DMA,
            pltpu.SemaphoreType.DMA,
        ],
        compiler_params=pltpu.CompilerParams(use_tc_tiling_on_sc=False),
    )
    def _scatter(msg_hbm, key_hbm, zer_hbm, out_hbm, idx_v, msg_a, msg_b,
                 acc, sem_a, sem_b):
        cid = lax.axis_index("c")
        sid = lax.axis_index("s")
        r0 = sid * ROWS_SUB
        e0 = sid * (E // NSUB)

        def _one_pass(mi, ki, p):
            pltpu.sync_copy(zer_hbm.at[pl.ds(r0, ROWS_SUB)],
                            acc.at[pl.ds(r0, ROWS_SUB)])
            pltpu.sync_copy(key_hbm.at[ki].at[sid], idx_v)
            plsc.subcore_barrier()

            mh = msg_hbm.at[mi]
            pltpu.async_copy(mh.at[pl.ds(e0, CHUNK)], msg_a, sem_a)

            def body(j2, carry):
                j = 2 * j2
                d1 = pltpu.async_copy(
                    mh.at[pl.ds(e0 + (j + 1) * CHUNK, CHUNK)], msg_b, sem_b)
                pltpu.make_async_copy(mh.at[pl.ds(0, CHUNK)], msg_a,
                                      sem_a).wait()
                pltpu.sync_copy(msg_a, acc.at[idx_v.at[j]], add=True)

                @pl.when(j2 + 1 < SC_CHUNKS // 2)
                def _():
                    pltpu.async_copy(
                        mh.at[pl.ds(e0 + (j + 2) * CHUNK, CHUNK)], msg_a,
                        sem_a)

                d1.wait()
                pltpu.sync_copy(msg_b, acc.at[idx_v.at[j + 1]], add=True)
                return carry

            lax.fori_loop(0, SC_CHUNKS // 2, body, 0)
            plsc.subcore_barrier()
            pltpu.sync_copy(acc.at[pl.ds(r0, ROWS_SUB)],
                            out_hbm.at[p].at[pl.ds(r0, ROWS_SUB)])
            plsc.subcore_barrier()

        for p, (mi, ki, co) in enumerate(cfg):
            pl.when(cid == co)(
                lambda mi=mi, ki=ki, p=p: _one_pass(mi, ki, p))

    return _scatter


@functools.lru_cache(maxsize=None)
def _get_scatter(kind):
    if kind == "gmp":
        return _make_scatter(GM_CFG, 3, 2)
    return _make_scatter(CV_CFG, 3, 1)


def _make_gather2():
    mesh = plsc.VectorSubcoreMesh(core_axis_name="c", subcore_axis_name="s")

    @functools.partial(
        pl.kernel,
        out_type=(jax.ShapeDtypeStruct((E, CH), _F32),
                  jax.ShapeDtypeStruct((E, CH), _F32)),
        mesh=mesh,
        scratch_types=[
            pltpu.VMEM((SC_CHUNKS, CHUNK), jnp.int32),
            pltpu.VMEM((CHUNK, CH), _F32),
            pltpu.VMEM((CHUNK, CH), _F32),
            pltpu.SemaphoreType.DMA,
            pltpu.SemaphoreType.DMA,
            pltpu.SemaphoreType.DMA,
            pltpu.SemaphoreType.DMA,
        ],
        compiler_params=pltpu.CompilerParams(use_tc_tiling_on_sc=False),
    )
    def _gather2(xl_hbm, xr_hbm, sidx_hbm, didx_hbm, oxl_hbm, oxr_hbm,
                 idx_v, ra_v, rb_v, gsem_a, gsem_b, osem_a, osem_b):
        cid = lax.axis_index("c")
        sid = lax.axis_index("s")
        e0 = sid * (E // NSUB)

        def _chain(tab_hbm, keys_hbm, out_hbm):
            pltpu.sync_copy(keys_hbm.at[sid], idx_v)

            def body(j2, carry):
                j = 2 * j2

                @pl.when(j2 > 0)
                def _():
                    pltpu.make_async_copy(
                        ra_v, out_hbm.at[pl.ds(e0, CHUNK)], osem_a).wait()
                    pltpu.make_async_copy(
                        rb_v, out_hbm.at[pl.ds(e0, CHUNK)], osem_b).wait()

                ga = pltpu.async_copy(tab_hbm.at[idx_v.at[j]], ra_v, gsem_a)
                gb = pltpu.async_copy(tab_hbm.at[idx_v.at[j + 1]], rb_v,
                                      gsem_b)
                ga.wait()
                pltpu.async_copy(ra_v, out_hbm.at[pl.ds(e0 + j * CHUNK,
                                                        CHUNK)], osem_a)
                gb.wait()
                pltpu.async_copy(rb_v, out_hbm.at[pl.ds(e0 + (j + 1) * CHUNK,
                                                        CHUNK)], osem_b)
                return carry

            lax.fori_loop(0, SC_CHUNKS // 2, body, 0)
            pltpu.make_async_copy(ra_v, out_hbm.at[pl.ds(e0, CHUNK)],
                                  osem_a).wait()
            pltpu.make_async_copy(rb_v, out_hbm.at[pl.ds(e0, CHUNK)],
                                  osem_b).wait()

        pl.when(cid == 0)(lambda: _chain(xl_hbm, sidx_hbm, oxl_hbm))
        pl.when(cid == 1)(lambda: _chain(xr_hbm, didx_hbm, oxr_hbm))

    return _gather2


@functools.lru_cache(maxsize=None)
def _get_gather2():
    return _make_gather2()


def _sc_gather2(xl, xr, sidx, didx):
    return _get_gather2()(xl, xr, sidx, didx)


def _sc_scatter(msg, keys, kind):
    zer = jnp.zeros((N, 128), _F32)
    return _get_scatter(kind)(msg, keys, zer)


def kernel(left_x, right_x, left_edge_index, right_edge_index, edge_tokens,
           edge_rest, embed_table, lstm_params, conv_params):
    emb_pad = jnp.zeros((32, 32), _F32).at[:30].set(embed_table)
    pwf = lstm_params['W_ih_f'].T
    whh = lstm_params['W_hh_f'].T
    bf = (lstm_params['b_ih_f'] + lstm_params['b_hh_f']).reshape(1, 256)
    pwb = lstm_params['W_ih_b'].T
    whb = lstm_params['W_hh_b'].T
    bb = (lstm_params['b_ih_b'] + lstm_params['b_hh_b']).reshape(1, 256)
    regex, gmsg = _lstm_call(edge_tokens, edge_rest, emb_pad, pwf, whh, bf,
                             pwb, whb, bb)

    s = left_edge_index[0]
    t = left_edge_index[1]
    gm_keys = jnp.stack([s.reshape(NSUB, SC_CHUNKS, CHUNK),
                         t.reshape(NSUB, SC_CHUNKS, CHUNK)], axis=0)
    gm = _sc_scatter(gmsg, gm_keys, "gmp")

    p0 = conv_params[0]
    xl_l, xr_l, xl_r, xr_r = _prep0_call(gm, left_x, right_x, p0['Wl'],
                                         p0['Wr'])

    def chain(xl, xr, edge_index):
        src = edge_index[0]
        dst = edge_index[1]
        sidx = src.reshape(NSUB, SC_CHUNKS, CHUNK)
        didx = dst.reshape(NSUB, SC_CHUNKS, CHUNK)
        cv_keys = didx[None]
        for li, p in enumerate(conv_params):
            xls, xrd = _sc_gather2(xl, xr, sidx, didx)
            msg = _edge_call(xls, xrd, regex, p['We'],
                             p['att'].reshape(1, CH))
            nd = _sc_scatter(msg, cv_keys, "conv")
            b = p['b'].reshape(1, CH)
            if li + 1 < len(conv_params):
                pn = conv_params[li + 1]
                xl, xr = _finish_call(nd, b, pn['Wl'], pn['Wr'])
            else:
                return _finish_call(nd, b)

    out_l = chain(xl_l, xr_l, left_edge_index)
    out_r = chain(xl_r, xr_r, right_edge_index)
    return jnp.concatenate([out_l, out_r], axis=-1)
